# baseline (speedup 1.0000x reference)
import jax
import jax.numpy as jnp
from jax import lax
from jax.experimental import pallas as pl
from jax.experimental.pallas import tpu as pltpu
from jax.experimental.pallas import tpu_sc as plsc

N = 10000
D = 128
E = 320000
NC = 2
NS = 16
NW = NC * NS
EPT = E // NW
CE = 64
EPAD = 10240
NCH = EPAD // CE
CH = 128
NPAD = 10240
STRIPE = NPAD // NS
KD = STRIPE // CH
BN = 512


def _drain(out, iot, r0, r1, acc, orow, gsem):
    def dk(k, carry):
        pltpu.async_copy(acc.at[iot.at[k]], r0.at[pl.ds(0, CH)], gsem).wait()
        pltpu.sync_copy(r0.at[pl.ds(0, CH)], out.at[pl.ds(orow + k * CH, CH)])
        return carry
    lax.fori_loop(0, KD, dk, 0)


def _seg_body(h_hbm, src_hbm, dst_hbm, zeros_hbm, ones_hbm, iota_hbm,
              out_sum, out_cnt,
              sidx0, sidx1, didx0, didx1, iot, rows0, rows1, onesr,
              acc, isem, gsem, ssem):
    c = lax.axis_index("c")
    s = lax.axis_index("s")
    wid = c * NS + s
    ebase = wid * EPAD
    orow = wid * STRIPE
    sidx = [sidx0, sidx1]
    didx = [didx0, didx1]
    rows = [rows0, rows1]

    pltpu.sync_copy(iota_hbm.at[pl.ds(s * 8, 8)], iot)
    pltpu.sync_copy(zeros_hbm, rows0.at[pl.ds(0, CH)])
    pltpu.sync_copy(ones_hbm, onesr)

    def zk(k, carry):
        pltpu.sync_copy(rows0.at[pl.ds(0, CH)], acc.at[iot.at[k]])
        return carry
    lax.fori_loop(0, KD, zk, 0)
    plsc.subcore_barrier()

    pltpu.async_copy(src_hbm.at[pl.ds(ebase, CE)], sidx0, isem.at[0])
    pltpu.async_copy(dst_hbm.at[pl.ds(ebase, CE)], didx0, isem.at[0])

    def fstep(g, carry):
      for b in range(2):
        j = g * 2 + b
        nb = 1 - b
        pltpu.make_async_copy(src_hbm.at[pl.ds(ebase, CE)], sidx[b],
                              isem.at[b]).wait()
        pltpu.make_async_copy(dst_hbm.at[pl.ds(ebase, CE)], didx[b],
                              isem.at[b]).wait()
        pltpu.async_copy(h_hbm.at[sidx[b]], rows[b], gsem)

        @pl.when(j >= 1)
        def _():
            pltpu.make_async_copy(rows[nb], acc.at[didx[nb]], ssem).wait()

        @pl.when(j + 1 < NCH)
        def _():
            eb = ebase + (j + 1) * CE
            pltpu.async_copy(src_hbm.at[pl.ds(eb, CE)], sidx[nb], isem.at[nb])
            pltpu.async_copy(dst_hbm.at[pl.ds(eb, CE)], didx[nb], isem.at[nb])

        pltpu.make_async_copy(h_hbm.at[sidx[b]], rows[b], gsem).wait()
        pltpu.async_copy(rows[b], acc.at[didx[b]], ssem, add=True)
      return carry

    lax.fori_loop(0, NCH // 2, fstep, 0)
    pltpu.make_async_copy(rows0, acc.at[didx0], ssem).wait()
    plsc.subcore_barrier()
    _drain(out_sum, iot, rows0, rows1, acc, orow, gsem)
    plsc.subcore_barrier()

    pltpu.sync_copy(zeros_hbm, rows0.at[pl.ds(0, CH)])
    lax.fori_loop(0, KD, zk, 0)
    plsc.subcore_barrier()

    pltpu.async_copy(dst_hbm.at[pl.ds(ebase, CE)], didx0, isem.at[0])

    def cstep(g, carry):
      for b in range(2):
        j = g * 2 + b
        nb = 1 - b
        pltpu.make_async_copy(dst_hbm.at[pl.ds(ebase, CE)], didx[b],
                              isem.at[b]).wait()

        @pl.when(j >= 1)
        def _():
            pltpu.make_async_copy(onesr, acc.at[didx[nb]], ssem).wait()

        @pl.when(j + 1 < NCH)
        def _():
            eb = ebase + (j + 1) * CE
            pltpu.async_copy(dst_hbm.at[pl.ds(eb, CE)], didx[nb], isem.at[nb])

        pltpu.async_copy(onesr, acc.at[didx[b]], ssem, add=True)
      return carry

    lax.fori_loop(0, NCH // 2, cstep, 0)
    pltpu.make_async_copy(onesr, acc.at[didx0], ssem).wait()
    plsc.subcore_barrier()
    _drain(out_cnt, iot, rows0, rows1, acc, orow, gsem)


_seg_call = pl.kernel(
    _seg_body,
    out_type=(jax.ShapeDtypeStruct((NC * NPAD, D), jnp.float32),
              jax.ShapeDtypeStruct((NC * NPAD, D), jnp.float32)),
    mesh=plsc.VectorSubcoreMesh(core_axis_name="c", subcore_axis_name="s",
                                num_cores=2),
    scratch_types=[
        pltpu.VMEM((CE,), jnp.int32),
        pltpu.VMEM((CE,), jnp.int32),
        pltpu.VMEM((CE,), jnp.int32),
        pltpu.VMEM((CE,), jnp.int32),
        pltpu.VMEM((8, CH), jnp.int32),
        pltpu.VMEM((CE, D), jnp.float32),
        pltpu.VMEM((CE, D), jnp.float32),
        pltpu.VMEM((CE, D), jnp.float32),
        pltpu.VMEM_SHARED((NPAD, D), jnp.float32),
        pltpu.SemaphoreType.DMA((2,)),
        pltpu.SemaphoreType.DMA,
        pltpu.SemaphoreType.DMA,
    ],
)


def _dense_body(sum_ref, cnt_ref, h_ref, wl_ref, wr_ref, b_ref, o_ref):
    ssum = sum_ref[0] + sum_ref[1]
    cnt = cnt_ref[0, :, 0:1] + cnt_ref[1, :, 0:1]
    agg = ssum / jnp.maximum(cnt, 1.0)
    o = (jnp.dot(agg, wl_ref[...], preferred_element_type=jnp.float32)
         + jnp.dot(h_ref[...], wr_ref[...], preferred_element_type=jnp.float32)
         + b_ref[...])
    o_ref[...] = jnp.maximum(o, 0.0)


_dense_call = pl.pallas_call(
    _dense_body,
    grid=(NPAD // BN,),
    in_specs=[
        pl.BlockSpec((NC, BN, D), lambda i: (0, i, 0)),
        pl.BlockSpec((NC, BN, D), lambda i: (0, i, 0)),
        pl.BlockSpec((BN, D), lambda i: (i, 0)),
        pl.BlockSpec((D, D), lambda i: (0, 0)),
        pl.BlockSpec((D, D), lambda i: (0, 0)),
        pl.BlockSpec((1, D), lambda i: (0, 0)),
    ],
    out_specs=pl.BlockSpec((BN, D), lambda i: (i, 0)),
    out_shape=jax.ShapeDtypeStruct((NPAD, D), jnp.float32),
)


@jax.jit
def kernel(x, edge_indices, W_l0, b_l0, W_r0, W_l1, b_l1, W_r1):
    zeros = jnp.zeros((CH, D), jnp.float32)
    ones = jnp.ones((CE, D), jnp.float32)
    iota3 = jnp.pad(
        jnp.arange(NPAD, dtype=jnp.int32).reshape(NS, KD, CH),
        ((0, 0), (0, 8 - KD), (0, 0)),
        constant_values=NPAD - 1).reshape(NS * 8, CH)
    pad_e = EPAD - EPT
    h = jnp.pad(x, ((0, NPAD - N), (0, 0)))
    for Wl, bl, Wr, ei in ((W_l0, b_l0, W_r0, edge_indices[0]),
                           (W_l1, b_l1, W_r1, edge_indices[1])):
        src1 = jnp.concatenate(
            [ei[0].reshape(NW, EPT),
             jnp.zeros((NW, pad_e), jnp.int32)], axis=1).reshape(NW * EPAD)
        dst1 = jnp.concatenate(
            [ei[1].reshape(NW, EPT),
             jnp.full((NW, pad_e), NPAD - 1, jnp.int32)],
            axis=1).reshape(NW * EPAD)
        psum, pcnt = _seg_call(h, src1, dst1, zeros, ones, iota3)
        h = _dense_call(psum.reshape(NC, NPAD, D), pcnt.reshape(NC, NPAD, D),
                        h, Wl.T, Wr.T, bl.reshape(1, D))
    return h[:N, :D]


# speedup vs baseline: 1.4543x; 1.4543x over previous
import jax
import jax.numpy as jnp
from jax import lax
from jax.experimental import pallas as pl
from jax.experimental.pallas import tpu as pltpu
from jax.experimental.pallas import tpu_sc as plsc

N = 10000
D = 128
E = 320000
NC = 2
NS = 16
NW = NC * NS
EPT = E // NW
CH = 128
NFULL = EPT // CH
REM = EPT - NFULL * CH
NPAD = 10240
STRIPE = NPAD // NS
BN = 512


def _zero_acc(iota_hbm, didx, rows, acc, srow):
    for k in range(STRIPE // CH):
        pltpu.sync_copy(iota_hbm.at[pl.ds(srow + k * CH, CH)], didx)
        pltpu.sync_copy(rows, acc.at[didx])


def _drain_acc(iota_hbm, didx, rows, acc, out, srow, orow, sem):
    for k in range(STRIPE // CH):
        pltpu.sync_copy(iota_hbm.at[pl.ds(srow + k * CH, CH)], didx)
        pltpu.async_copy(acc.at[didx], rows, sem).wait()
        pltpu.sync_copy(rows, out.at[pl.ds(orow + k * CH, CH)])


def _seg_body(h_hbm, src_hbm, dst_hbm, zeros_hbm, ones_hbm, iota_hbm,
              out_sum, out_cnt,
              sidx, didx, sidx2, didx2, rows, rows2, onesr, acc, sem):
    c = lax.axis_index("c")
    s = lax.axis_index("s")
    wid = c * NS + s
    tile_base = wid * EPT
    srow = s * STRIPE
    orow = wid * STRIPE

    pltpu.sync_copy(zeros_hbm, rows)
    pltpu.sync_copy(ones_hbm, onesr)
    _zero_acc(iota_hbm, didx, rows, acc, srow)
    plsc.subcore_barrier()

    def chunk(j, carry):
        eb = tile_base + j * CH
        pltpu.sync_copy(src_hbm.at[pl.ds(eb, CH)], sidx)
        pltpu.sync_copy(dst_hbm.at[pl.ds(eb, CH)], didx)
        pltpu.async_copy(h_hbm.at[sidx], rows, sem).wait()
        pltpu.sync_copy(rows, acc.at[didx], add=True)
        return carry

    lax.fori_loop(0, NFULL, chunk, 0)

    eb = tile_base + NFULL * CH
    pltpu.sync_copy(src_hbm.at[pl.ds(eb, REM)], sidx2)
    pltpu.sync_copy(dst_hbm.at[pl.ds(eb, REM)], didx2)
    pltpu.async_copy(h_hbm.at[sidx2], rows2, sem).wait()
    pltpu.sync_copy(rows2, acc.at[didx2], add=True)

    plsc.subcore_barrier()
    _drain_acc(iota_hbm, didx, rows, acc, out_sum, srow, orow, sem)
    plsc.subcore_barrier()

    pltpu.sync_copy(zeros_hbm, rows)
    _zero_acc(iota_hbm, didx, rows, acc, srow)
    plsc.subcore_barrier()

    def cchunk(j, carry):
        eb2 = tile_base + j * CH
        pltpu.sync_copy(dst_hbm.at[pl.ds(eb2, CH)], didx)
        pltpu.sync_copy(onesr, acc.at[didx], add=True)
        return carry

    lax.fori_loop(0, NFULL, cchunk, 0)

    eb = tile_base + NFULL * CH
    pltpu.sync_copy(dst_hbm.at[pl.ds(eb, REM)], didx2)
    pltpu.sync_copy(onesr.at[pl.ds(0, REM)], acc.at[didx2], add=True)

    plsc.subcore_barrier()
    _drain_acc(iota_hbm, didx, rows, acc, out_cnt, srow, orow, sem)


_seg_call = pl.kernel(
    _seg_body,
    out_type=(jax.ShapeDtypeStruct((NC * NPAD, D), jnp.float32),
              jax.ShapeDtypeStruct((NC * NPAD, D), jnp.float32)),
    mesh=plsc.VectorSubcoreMesh(core_axis_name="c", subcore_axis_name="s",
                                num_cores=2),
    scratch_types=[
        pltpu.VMEM((CH,), jnp.int32),
        pltpu.VMEM((CH,), jnp.int32),
        pltpu.VMEM((REM,), jnp.int32),
        pltpu.VMEM((REM,), jnp.int32),
        pltpu.VMEM((CH, D), jnp.float32),
        pltpu.VMEM((REM, D), jnp.float32),
        pltpu.VMEM((CH, D), jnp.float32),
        pltpu.VMEM_SHARED((NPAD, D), jnp.float32),
        pltpu.SemaphoreType.DMA,
    ],
)


def _dense_body(sum_ref, cnt_ref, h_ref, wl_ref, wr_ref, b_ref, o_ref):
    ssum = sum_ref[0] + sum_ref[1]
    cnt = cnt_ref[0, :, 0:1] + cnt_ref[1, :, 0:1]
    agg = ssum / jnp.maximum(cnt, 1.0)
    o = (jnp.dot(agg, wl_ref[...], preferred_element_type=jnp.float32)
         + jnp.dot(h_ref[...], wr_ref[...], preferred_element_type=jnp.float32)
         + b_ref[...])
    o_ref[...] = jnp.maximum(o, 0.0)


_dense_call = pl.pallas_call(
    _dense_body,
    grid=(NPAD // BN,),
    in_specs=[
        pl.BlockSpec((NC, BN, D), lambda i: (0, i, 0)),
        pl.BlockSpec((NC, BN, D), lambda i: (0, i, 0)),
        pl.BlockSpec((BN, D), lambda i: (i, 0)),
        pl.BlockSpec((D, D), lambda i: (0, 0)),
        pl.BlockSpec((D, D), lambda i: (0, 0)),
        pl.BlockSpec((1, D), lambda i: (0, 0)),
    ],
    out_specs=pl.BlockSpec((BN, D), lambda i: (i, 0)),
    out_shape=jax.ShapeDtypeStruct((NPAD, D), jnp.float32),
)


@jax.jit
def kernel(x, edge_indices, W_l0, b_l0, W_r0, W_l1, b_l1, W_r1):
    zeros = jnp.zeros((CH, D), jnp.float32)
    ones = jnp.ones((CH, D), jnp.float32)
    iota = jnp.arange(NPAD, dtype=jnp.int32)
    h = jnp.pad(x, ((0, NPAD - N), (0, 0)))
    for Wl, bl, Wr, ei in ((W_l0, b_l0, W_r0, edge_indices[0]),
                           (W_l1, b_l1, W_r1, edge_indices[1])):
        psum, pcnt = _seg_call(h, ei[0], ei[1], zeros, ones, iota)
        h = _dense_call(psum.reshape(NC, NPAD, D), pcnt.reshape(NC, NPAD, D),
                        h, Wl.T, Wr.T, bl.reshape(1, D))
    return h[:N, :D]
